# baseline (speedup 1.0000x reference)
import functools

import jax
import jax.numpy as jnp
from jax import lax
from jax.experimental import pallas as pl
from jax.experimental.pallas import tpu as pltpu
from jax.experimental.pallas import tpu_sc as plsc

_SIG2 = 0.05 * 0.05
_SIGV2 = 0.1 * 0.1

_BQ = 2048
_BN = 4096
_QPW = 256


_CH = 1024


def _nn_body(q_ref, pt_ref, pn_ref, o_ref, bv_ref, bs_ref):
    j = pl.program_id(1)
    nslab = _BN // 128
    q = q_ref[...]
    qq = jnp.sum(q * q, axis=1, keepdims=True)
    pn = pn_ref[...]

    @pl.when(j == 0)
    def _():
        bv_ref[...] = jnp.full((_BQ, 128), jnp.inf, jnp.float32)
        bs_ref[...] = jnp.zeros((_BQ, 128), jnp.int32)

    acc = None
    for c in range(_BN // _CH):
        qp = lax.dot_general(
            q, pt_ref[c * _CH:(c + 1) * _CH, :],
            dimension_numbers=(((1,), (1,)), ((), ())),
            preferred_element_type=jnp.float32)
        qp2 = -2.0 * qp
        lvl = []
        for s in range(_CH // 128):
            g = c * (_CH // 128) + s
            d = ((qq + pn[:, g * 128:(g + 1) * 128])
                 + qp2[:, s * 128:(s + 1) * 128])
            lvl.append((d, jnp.int32(j * nslab + g)))
        while len(lvl) > 1:
            nxt = []
            for t in range(0, len(lvl), 2):
                (va, ia), (vb, ib) = lvl[t], lvl[t + 1]
                lt = vb < va
                nxt.append((jnp.where(lt, vb, va), jnp.where(lt, ib, ia)))
            lvl = nxt
        cv, ci = lvl[0]
        if acc is None:
            acc = (cv, ci)
        else:
            av, ai = acc
            lt = cv < av
            acc = (jnp.where(lt, cv, av), jnp.where(lt, ci, ai))
    av, ai = acc
    lt = av < bv_ref[...]
    bv_ref[...] = jnp.where(lt, av, bv_ref[...])
    bs_ref[...] = jnp.where(lt, ai, bs_ref[...])

    @pl.when(j == pl.num_programs(1) - 1)
    def _():
        bv = bv_ref[...]
        bs = bs_ref[...]
        m = jnp.min(bv, axis=1, keepdims=True)
        lane = lax.broadcasted_iota(jnp.int32, (_BQ, 128), 1)
        n_idx = bs * 128 + lane
        o_ref[...] = jnp.min(
            jnp.where(bv == m, n_idx, jnp.int32(2**30)), axis=1,
            keepdims=True)


def _nearest_pallas(query, points):
    b = query.shape[0]
    n = points.shape[0]
    npad = ((n + _BN - 1) // _BN) * _BN
    pts = jnp.concatenate(
        [points, jnp.full((npad - n, 3), 100.0, jnp.float32)], axis=0)
    pn = jnp.sum(pts * pts, axis=1)[None, :]
    grid = (b // _BQ, npad // _BN)
    nn = pl.pallas_call(
        _nn_body,
        grid=grid,
        in_specs=[
            pl.BlockSpec((_BQ, 3), lambda i, j: (i, 0)),
            pl.BlockSpec((_BN, 3), lambda i, j: (j, 0)),
            pl.BlockSpec((1, _BN), lambda i, j: (0, j)),
        ],
        out_specs=pl.BlockSpec((_BQ, 1), lambda i, j: (i, 0)),
        out_shape=jax.ShapeDtypeStruct((b, 1), jnp.int32),
        scratch_shapes=[
            pltpu.VMEM((_BQ, 128), jnp.float32),
            pltpu.VMEM((_BQ, 128), jnp.int32),
        ],
    )(query, pts, pn)
    return nn[:, 0]


def _idw_body(qpw, q_hbm, pf_hbm, a_hbm, adj_hbm, off_hbm, nn_hbm, out_hbm,
              qbuf, nn128, nn3, offs, offs2, refv, ctrx, ctry, ctrz, w0buf,
              abuf, vbuf, pxb, pyb, pzb, obuf, sem_a, sem_b, sem_c, sem_d):
    wid = lax.axis_index("s") * 2 + lax.axis_index("c")
    base = wid * qpw
    iota16 = lax.broadcasted_iota(jnp.int32, (16,), 0)

    ch = min(128, qpw)
    pltpu.sync_copy(q_hbm.at[pl.ds(base * 3, qpw * 3)], qbuf)
    for j in range(qpw // ch):
        sl = pl.ds(j * ch, ch)
        pltpu.sync_copy(nn_hbm.at[pl.ds(base + j * ch, ch)], nn128)
        for g in range(ch // 16):
            s = pl.ds(g * 16, 16)
            nn3[s] = nn128[s] + 1
        c1 = pltpu.async_copy(off_hbm.at[nn128], offs.at[sl], sem_a)
        c2 = pltpu.async_copy(off_hbm.at[nn3], offs2.at[sl], sem_b)
        c3 = pltpu.async_copy(a_hbm.at[nn128], refv.at[sl], sem_c)
        c1.wait()
        c2.wait()
        c3.wait()
        for comp, dst in ((0, ctrx), (1, ctry), (2, ctrz)):
            for g in range(ch // 16):
                s = pl.ds(g * 16, 16)
                nn3[s] = nn128[s] * 3 + comp
            pltpu.async_copy(pf_hbm.at[nn3], dst.at[sl], sem_a).wait()

    for g in range(qpw // 16):
        rows3 = (iota16 + g * 16) * 3
        s = pl.ds(g * 16, 16)
        qx = plsc.load_gather(qbuf, [rows3])
        qy = plsc.load_gather(qbuf, [rows3 + 1])
        qz = plsc.load_gather(qbuf, [rows3 + 2])
        dx = qx - ctrx[s]
        dy = qy - ctry[s]
        dz = qz - ctrz[s]
        d0 = dx * dx + dy * dy + dz * dz
        w0buf[s] = jnp.exp(-d0 / _SIG2) + 1e-6

    def q_body(qi, carry):
        qsel = iota16 * 0 + qi
        off = plsc.load_gather(offs, [qsel])[0]
        cnt = plsc.load_gather(offs2, [qsel])[0] - off
        rv = plsc.load_gather(refv, [qsel])[0]
        qxs = plsc.load_gather(qbuf, [qsel * 3])[0]
        qys = plsc.load_gather(qbuf, [qsel * 3 + 1])[0]
        qzs = plsc.load_gather(qbuf, [qsel * 3 + 2])[0]

        def cond(c):
            return c[0] < cnt

        def body(c):
            k, aw, av = c
            off_k = off + k
            al = (off_k // 8) * 8
            sh = off_k - al
            pltpu.sync_copy(adj_hbm.at[pl.ds(al, 24)], abuf)
            idxv = plsc.load_gather(abuf, [iota16 + sh])
            idx3 = idxv * 3
            cv = pltpu.async_copy(a_hbm.at[idxv], vbuf, sem_a)
            cx = pltpu.async_copy(pf_hbm.at[idx3], pxb, sem_b)
            cy = pltpu.async_copy(pf_hbm.at[idx3 + 1], pyb, sem_c)
            cz = pltpu.async_copy(pf_hbm.at[idx3 + 2], pzb, sem_d)
            cv.wait()
            cx.wait()
            cy.wait()
            cz.wait()
            vals = vbuf[...]
            px = pxb[...]
            py = pyb[...]
            pz = pzb[...]
            dx = qxs - px
            dy = qys - py
            dz = qzs - pz
            dsq = dx * dx + dy * dy + dz * dz
            w = jnp.exp(-dsq / _SIG2)
            vd = vals - rv
            w = w * jnp.exp(-vd * vd / _SIGV2)
            valid = (iota16 + k) < cnt
            w = jnp.where(valid, w, 0.0) + valid.astype(jnp.float32) * 1e-6
            mv = jnp.where(valid, vals, 0.0)
            return (k + 16, aw + w, av + w * mv)

        z16 = jnp.zeros((16,), jnp.float32)
        _, aw, av = lax.while_loop(cond, body, (jnp.int32(0), z16, z16))
        w0 = plsc.load_gather(w0buf, [qsel])[0]
        tw = w0 + jnp.sum(aw)
        tv = w0 * rv + jnp.sum(av)
        z16f = jnp.zeros((16,), jnp.float32)
        res = (z16f + tv) / (z16f + tw)
        plsc.store_scatter(obuf, [qsel], res, mask=iota16 == 0)
        return carry

    lax.fori_loop(0, qpw, q_body, 0)
    pltpu.sync_copy(obuf, out_hbm.at[pl.ds(base, qpw)])


def _idw_pallas(query, points, activated, adj_pad, adj_off, nn_idx):
    b = query.shape[0]
    qpw = b // 32
    mesh = plsc.VectorSubcoreMesh(core_axis_name="c", subcore_axis_name="s")
    fn = pl.kernel(
        functools.partial(_idw_body, qpw),
        out_type=jax.ShapeDtypeStruct((b,), jnp.float32),
        mesh=mesh,
        compiler_params=pltpu.CompilerParams(needs_layout_passes=False),
        scratch_types=[
            pltpu.VMEM((qpw * 3,), jnp.float32),
            pltpu.VMEM((min(128, qpw),), jnp.int32),
            pltpu.VMEM((min(128, qpw),), jnp.int32),
            pltpu.VMEM((qpw,), jnp.int32),
            pltpu.VMEM((qpw,), jnp.int32),
            pltpu.VMEM((qpw,), jnp.float32),
            pltpu.VMEM((qpw,), jnp.float32),
            pltpu.VMEM((qpw,), jnp.float32),
            pltpu.VMEM((qpw,), jnp.float32),
            pltpu.VMEM((qpw,), jnp.float32),
            pltpu.VMEM((24,), jnp.int32),
            pltpu.VMEM((16,), jnp.float32),
            pltpu.VMEM((16,), jnp.float32),
            pltpu.VMEM((16,), jnp.float32),
            pltpu.VMEM((16,), jnp.float32),
            pltpu.VMEM((qpw,), jnp.float32),
            pltpu.SemaphoreType.DMA,
            pltpu.SemaphoreType.DMA,
            pltpu.SemaphoreType.DMA,
            pltpu.SemaphoreType.DMA,
        ],
    )
    return fn(query.reshape(-1), points.reshape(-1), activated, adj_pad,
              adj_off, nn_idx)


@functools.partial(jax.jit)
def kernel(query, points, activated, adjacency, adjacency_offsets):
    adj_pad = jnp.concatenate(
        [adjacency, jnp.zeros((32,), adjacency.dtype)], axis=0)
    h = query.shape[0] // 2
    outs = []
    for q in (query[:h], query[h:]):
        nn_idx = _nearest_pallas(q, points)
        outs.append(_idw_pallas(q, points, activated, adj_pad,
                                adjacency_offsets, nn_idx))
    return jnp.concatenate(outs, axis=0)


# speedup vs baseline: 1.1089x; 1.1089x over previous
import functools

import jax
import jax.numpy as jnp
from jax import lax
from jax.experimental import pallas as pl
from jax.experimental.pallas import tpu as pltpu
from jax.experimental.pallas import tpu_sc as plsc

_SIG2 = 0.05 * 0.05
_SIGV2 = 0.1 * 0.1

_BQ = 2048
_BN = 4096
_QPW = 256


_CH = 1024


def _nn_body(q_ref, qm_ref, pt_ref, pn_ref, o_ref, bv_ref, bs_ref):
    j = pl.program_id(1)
    nslab = _BN // 128
    q = q_ref[...]
    qq = jnp.sum(q * q, axis=1, keepdims=True)
    pn = pn_ref[...]

    @pl.when(j == 0)
    def _():
        bv_ref[...] = jnp.full((_BQ, 128), jnp.inf, jnp.float32)
        bs_ref[...] = jnp.zeros((_BQ, 128), jnp.int32)

    acc = None
    for c in range(_BN // _CH):
        qp2 = jnp.dot(qm_ref[...], pt_ref[:, c * _CH:(c + 1) * _CH],
                      preferred_element_type=jnp.float32)
        lvl = []
        for s in range(_CH // 128):
            g = c * (_CH // 128) + s
            d = ((qq + pn[:, g * 128:(g + 1) * 128])
                 + qp2[:, s * 128:(s + 1) * 128])
            lvl.append((d, jnp.int32(j * nslab + g)))
        while len(lvl) > 1:
            nxt = []
            for t in range(0, len(lvl), 2):
                (va, ia), (vb, ib) = lvl[t], lvl[t + 1]
                lt = vb < va
                nxt.append((jnp.where(lt, vb, va), jnp.where(lt, ib, ia)))
            lvl = nxt
        cv, ci = lvl[0]
        if acc is None:
            acc = (cv, ci)
        else:
            av, ai = acc
            lt = cv < av
            acc = (jnp.where(lt, cv, av), jnp.where(lt, ci, ai))
    av, ai = acc
    lt = av < bv_ref[...]
    bv_ref[...] = jnp.where(lt, av, bv_ref[...])
    bs_ref[...] = jnp.where(lt, ai, bs_ref[...])

    @pl.when(j == pl.num_programs(1) - 1)
    def _():
        bv = bv_ref[...]
        bs = bs_ref[...]
        m = jnp.min(bv, axis=1, keepdims=True)
        lane = lax.broadcasted_iota(jnp.int32, (_BQ, 128), 1)
        n_idx = bs * 128 + lane
        o_ref[...] = jnp.min(
            jnp.where(bv == m, n_idx, jnp.int32(2**30)), axis=1,
            keepdims=True)


def _nearest_pallas(query, points):
    b = query.shape[0]
    n = points.shape[0]
    npad = ((n + _BN - 1) // _BN) * _BN
    pts = jnp.concatenate(
        [points, jnp.full((npad - n, 3), 100.0, jnp.float32)], axis=0)
    pn = jnp.sum(pts * pts, axis=1)[None, :]
    pt8 = jnp.concatenate([pts.T, jnp.zeros((5, npad), jnp.float32)], axis=0)
    q8 = jnp.concatenate([query, jnp.zeros((b, 5), jnp.float32)], axis=1)
    qm8 = -2.0 * q8
    grid = (b // _BQ, npad // _BN)
    nn = pl.pallas_call(
        _nn_body,
        grid=grid,
        in_specs=[
            pl.BlockSpec((_BQ, 8), lambda i, j: (i, 0)),
            pl.BlockSpec((_BQ, 8), lambda i, j: (i, 0)),
            pl.BlockSpec((8, _BN), lambda i, j: (0, j)),
            pl.BlockSpec((1, _BN), lambda i, j: (0, j)),
        ],
        out_specs=pl.BlockSpec((_BQ, 1), lambda i, j: (i, 0)),
        out_shape=jax.ShapeDtypeStruct((b, 1), jnp.int32),
        scratch_shapes=[
            pltpu.VMEM((_BQ, 128), jnp.float32),
            pltpu.VMEM((_BQ, 128), jnp.int32),
        ],
    )(q8, qm8, pt8, pn)
    return nn[:, 0]


def _idw_body(qpw, q_hbm, pf_hbm, a_hbm, adj_hbm, off_hbm, nn_hbm, out_hbm,
              qbuf, nn128, nn3, offs, offs2, refv, ctrx, ctry, ctrz, w0buf,
              abuf, vbuf, pxb, pyb, pzb, obuf, sem_a, sem_b, sem_c, sem_d):
    wid = lax.axis_index("s") * 2 + lax.axis_index("c")
    base = wid * qpw
    iota16 = lax.broadcasted_iota(jnp.int32, (16,), 0)

    ch = min(128, qpw)
    pltpu.sync_copy(q_hbm.at[pl.ds(base * 3, qpw * 3)], qbuf)
    for j in range(qpw // ch):
        sl = pl.ds(j * ch, ch)
        pltpu.sync_copy(nn_hbm.at[pl.ds(base + j * ch, ch)], nn128)
        for g in range(ch // 16):
            s = pl.ds(g * 16, 16)
            nn3[s] = nn128[s] + 1
        c1 = pltpu.async_copy(off_hbm.at[nn128], offs.at[sl], sem_a)
        c2 = pltpu.async_copy(off_hbm.at[nn3], offs2.at[sl], sem_b)
        c3 = pltpu.async_copy(a_hbm.at[nn128], refv.at[sl], sem_c)
        c1.wait()
        c2.wait()
        c3.wait()
        for comp, dst in ((0, ctrx), (1, ctry), (2, ctrz)):
            for g in range(ch // 16):
                s = pl.ds(g * 16, 16)
                nn3[s] = nn128[s] * 3 + comp
            pltpu.async_copy(pf_hbm.at[nn3], dst.at[sl], sem_a).wait()

    for g in range(qpw // 16):
        rows3 = (iota16 + g * 16) * 3
        s = pl.ds(g * 16, 16)
        qx = plsc.load_gather(qbuf, [rows3])
        qy = plsc.load_gather(qbuf, [rows3 + 1])
        qz = plsc.load_gather(qbuf, [rows3 + 2])
        dx = qx - ctrx[s]
        dy = qy - ctry[s]
        dz = qz - ctrz[s]
        d0 = dx * dx + dy * dy + dz * dz
        w0buf[s] = jnp.exp(-d0 / _SIG2) + 1e-6

    def q_body(qi, carry):
        qsel = iota16 * 0 + qi
        off = plsc.load_gather(offs, [qsel])[0]
        cnt = plsc.load_gather(offs2, [qsel])[0] - off
        rv = plsc.load_gather(refv, [qsel])[0]
        qxs = plsc.load_gather(qbuf, [qsel * 3])[0]
        qys = plsc.load_gather(qbuf, [qsel * 3 + 1])[0]
        qzs = plsc.load_gather(qbuf, [qsel * 3 + 2])[0]

        def cond(c):
            return c[0] < cnt

        def body(c):
            k, aw, av = c
            off_k = off + k
            al = (off_k // 8) * 8
            sh = off_k - al
            pltpu.sync_copy(adj_hbm.at[pl.ds(al, 24)], abuf)
            idxv = plsc.load_gather(abuf, [iota16 + sh])
            idx3 = idxv * 3
            cv = pltpu.async_copy(a_hbm.at[idxv], vbuf, sem_a)
            cx = pltpu.async_copy(pf_hbm.at[idx3], pxb, sem_b)
            cy = pltpu.async_copy(pf_hbm.at[idx3 + 1], pyb, sem_c)
            cz = pltpu.async_copy(pf_hbm.at[idx3 + 2], pzb, sem_d)
            cv.wait()
            cx.wait()
            cy.wait()
            cz.wait()
            vals = vbuf[...]
            px = pxb[...]
            py = pyb[...]
            pz = pzb[...]
            dx = qxs - px
            dy = qys - py
            dz = qzs - pz
            dsq = dx * dx + dy * dy + dz * dz
            w = jnp.exp(-dsq / _SIG2)
            vd = vals - rv
            w = w * jnp.exp(-vd * vd / _SIGV2)
            valid = (iota16 + k) < cnt
            w = jnp.where(valid, w, 0.0) + valid.astype(jnp.float32) * 1e-6
            mv = jnp.where(valid, vals, 0.0)
            return (k + 16, aw + w, av + w * mv)

        z16 = jnp.zeros((16,), jnp.float32)
        _, aw, av = lax.while_loop(cond, body, (jnp.int32(0), z16, z16))
        w0 = plsc.load_gather(w0buf, [qsel])[0]
        tw = w0 + jnp.sum(aw)
        tv = w0 * rv + jnp.sum(av)
        z16f = jnp.zeros((16,), jnp.float32)
        res = (z16f + tv) / (z16f + tw)
        plsc.store_scatter(obuf, [qsel], res, mask=iota16 == 0)
        return carry

    lax.fori_loop(0, qpw, q_body, 0)
    pltpu.sync_copy(obuf, out_hbm.at[pl.ds(base, qpw)])


def _idw_pallas(query, points, activated, adj_pad, adj_off, nn_idx):
    b = query.shape[0]
    qpw = b // 32
    mesh = plsc.VectorSubcoreMesh(core_axis_name="c", subcore_axis_name="s")
    fn = pl.kernel(
        functools.partial(_idw_body, qpw),
        out_type=jax.ShapeDtypeStruct((b,), jnp.float32),
        mesh=mesh,
        compiler_params=pltpu.CompilerParams(needs_layout_passes=False),
        scratch_types=[
            pltpu.VMEM((qpw * 3,), jnp.float32),
            pltpu.VMEM((min(128, qpw),), jnp.int32),
            pltpu.VMEM((min(128, qpw),), jnp.int32),
            pltpu.VMEM((qpw,), jnp.int32),
            pltpu.VMEM((qpw,), jnp.int32),
            pltpu.VMEM((qpw,), jnp.float32),
            pltpu.VMEM((qpw,), jnp.float32),
            pltpu.VMEM((qpw,), jnp.float32),
            pltpu.VMEM((qpw,), jnp.float32),
            pltpu.VMEM((qpw,), jnp.float32),
            pltpu.VMEM((24,), jnp.int32),
            pltpu.VMEM((16,), jnp.float32),
            pltpu.VMEM((16,), jnp.float32),
            pltpu.VMEM((16,), jnp.float32),
            pltpu.VMEM((16,), jnp.float32),
            pltpu.VMEM((qpw,), jnp.float32),
            pltpu.SemaphoreType.DMA,
            pltpu.SemaphoreType.DMA,
            pltpu.SemaphoreType.DMA,
            pltpu.SemaphoreType.DMA,
        ],
    )
    return fn(query.reshape(-1), points.reshape(-1), activated, adj_pad,
              adj_off, nn_idx)


@functools.partial(jax.jit)
def kernel(query, points, activated, adjacency, adjacency_offsets):
    adj_pad = jnp.concatenate(
        [adjacency, jnp.zeros((32,), adjacency.dtype)], axis=0)
    h = query.shape[0] // 2
    outs = []
    for q in (query[:h], query[h:]):
        nn_idx = _nearest_pallas(q, points)
        outs.append(_idw_pallas(q, points, activated, adj_pad,
                                adjacency_offsets, nn_idx))
    return jnp.concatenate(outs, axis=0)
